# baseline (speedup 1.0000x reference)
import functools

import jax
import jax.numpy as jnp
from jax import lax
from jax.experimental import pallas as pl
from jax.experimental.pallas import tpu as pltpu
from jax.experimental.pallas import tpu_sc as plsc

H, B, N, D = 3, 4, 2048, 2048
K = N // 2
HB = H * B
BN = 256
NBLK = N // BN
SIGN = -2 ** 31
L = 16
CHUNKS = N // L
UNROLL = 8


def _tc_body(x_ref, w_ref, xout_ref, score_out_ref, score_ref):
    hb = pl.program_id(0)
    jn = pl.program_id(1)

    xraw = x_ref[...]
    xout_ref[...] = xraw

    x = xraw[0].astype(jnp.bfloat16).astype(jnp.float32)
    w = w_ref[...].astype(jnp.bfloat16).astype(jnp.float32)
    s = jax.lax.dot_general(x, w, (((1,), (1,)), ((), ())),
                            preferred_element_type=jnp.float32)
    score_ref[pl.ds(hb, 1), pl.ds(jn * BN, BN)] = s.reshape(1, BN)

    @pl.when(jnp.logical_and(hb == HB - 1, jn == NBLK - 1))
    def _():
        score_out_ref[...] = score_ref[...]


def _tc_scores(x, W):
    return pl.pallas_call(
        _tc_body,
        grid=(HB, NBLK),
        in_specs=[
            pl.BlockSpec((1, BN, D), lambda i, j: (i, j, 0)),
            pl.BlockSpec((1, D), lambda i, j: (0, 0)),
        ],
        out_specs=[
            pl.BlockSpec((1, BN, D), lambda i, j: (i, j, 0)),
            pl.BlockSpec((HB, N), lambda i, j: (0, 0)),
        ],
        out_shape=[
            jax.ShapeDtypeStruct((HB, N, D), jnp.float32),
            jax.ShapeDtypeStruct((HB, N), jnp.float32),
        ],
        scratch_shapes=[pltpu.VMEM((HB, N), jnp.float32)],
        compiler_params=pltpu.CompilerParams(
            dimension_semantics=("arbitrary", "arbitrary")),
    )(x, W)


def _sc_select_body(scores_hbm, sel_hbm, row_v, key_v, sel_v):
    c = lax.axis_index("c")
    s = lax.axis_index("s")

    @pl.when(jnp.logical_and(c == 0, s < HB))
    def _select_row():
        pltpu.sync_copy(scores_hbm.at[s], row_v)

        def mk(i, _):
            for u in range(UNROLL):
                off = (i * UNROLL + u) * L
                sv = row_v[pl.ds(off, L)]
                sv = jnp.where(sv == 0.0, 0.0, sv)
                bv = lax.bitcast_convert_type(sv, jnp.int32)
                key_v[pl.ds(off, L)] = jnp.where(
                    bv >= 0, bv, bv ^ jnp.int32(0x7FFFFFFF))
            return 0
        lax.fori_loop(0, CHUNKS // UNROLL, mk, 0, unroll=False)

        sign = jnp.int32(SIGN)
        zeros = jnp.zeros((L,), jnp.int32)

        def count_pred(pred_of_chunk):
            def inner(j, av):
                for u in range(UNROLL):
                    off = (j * UNROLL + u) * L
                    av = av + plsc.all_reduce_population_count(
                        pred_of_chunk(off))
                return av
            return lax.fori_loop(0, CHUNKS // UNROLL, inner, zeros,
                                 unroll=False)

        def count_lt(t_s):
            return count_pred(lambda off: key_v[pl.ds(off, L)] < t_s)

        def bit_pass(i, acc):
            t = acc | lax.shift_left(jnp.int32(1), 31 - i)
            cnt = count_lt(t ^ sign)
            return jnp.where(cnt < K, t, acc)
        acc = lax.fori_loop(0, 32, bit_pass, zeros, unroll=False)
        vs = acc ^ sign
        r = K - count_lt(vs)

        lane = lax.iota(jnp.int32, L)

        def idx_pass(i, acc2):
            t = acc2 | lax.shift_left(jnp.int32(1), 10 - i)
            cnt = count_pred(
                lambda off: (key_v[pl.ds(off, L)] == vs) & ((lane + off) < t))
            return jnp.where(cnt < r, t, acc2)
        istar = lax.fori_loop(0, 11, idx_pass, zeros, unroll=False)

        def sel_pass(j, _):
            for u in range(UNROLL):
                off = (j * UNROLL + u) * L
                kv = key_v[pl.ds(off, L)]
                keep = (kv < vs) | ((kv == vs) & ((lane + off) <= istar))
                sel_v[pl.ds(off, L)] = jnp.where(keep, 1.0, 0.0)
            return 0
        lax.fori_loop(0, CHUNKS // UNROLL, sel_pass, 0, unroll=False)
        pltpu.sync_copy(sel_v, sel_hbm.at[s])


def _sc_combine_body(sel_hbm, src_hbm, out_hbm, h0_v, h1_v, h2_v, src_v, out_v):
    c = lax.axis_index("c")
    s = lax.axis_index("s")

    @pl.when(jnp.logical_and(c == 0, s < B))
    def _combine():
        pltpu.sync_copy(sel_hbm.at[s], h0_v)
        pltpu.sync_copy(sel_hbm.at[B + s], h1_v)
        pltpu.sync_copy(sel_hbm.at[2 * B + s], h2_v)
        pltpu.sync_copy(src_hbm.at[s], src_v)

        def comb(i, _):
            for u in range(UNROLL):
                off = (i * UNROLL + u) * L
                cnt = (h0_v[pl.ds(off, L)] + h1_v[pl.ds(off, L)]
                       + h2_v[pl.ds(off, L)])
                out_v[pl.ds(off, L)] = cnt + (float(H) - cnt) * src_v[pl.ds(off, L)]
            return 0
        lax.fori_loop(0, CHUNKS // UNROLL, comb, 0, unroll=False)
        pltpu.sync_copy(out_v, out_hbm.at[s])


_SC_MESH = plsc.VectorSubcoreMesh(core_axis_name="c", subcore_axis_name="s")

_sc_select = functools.partial(
    pl.kernel,
    out_type=jax.ShapeDtypeStruct((HB, N), jnp.float32),
    mesh=_SC_MESH,
    scratch_types=[
        pltpu.VMEM((N,), jnp.float32),
        pltpu.VMEM((N,), jnp.int32),
        pltpu.VMEM((N,), jnp.float32),
    ],
    compiler_params=pltpu.CompilerParams(needs_layout_passes=False),
)(_sc_select_body)

_sc_combine = functools.partial(
    pl.kernel,
    out_type=jax.ShapeDtypeStruct((B, N), jnp.float32),
    mesh=_SC_MESH,
    scratch_types=[
        pltpu.VMEM((N,), jnp.float32),
        pltpu.VMEM((N,), jnp.float32),
        pltpu.VMEM((N,), jnp.float32),
        pltpu.VMEM((N,), jnp.float32),
        pltpu.VMEM((N,), jnp.float32),
    ],
    compiler_params=pltpu.CompilerParams(needs_layout_passes=False),
)(_sc_combine_body)


def kernel(x_list, src_mask, W, b):
    x = x_list.reshape(HB, N, D)
    src = src_mask.reshape(B, N)
    x_out, scores = _tc_scores(x, W)
    sel = _sc_select(scores)
    mask = _sc_combine(sel, src)
    return x_out.reshape(H, B, N, D), mask.reshape(B, 1, N)


# speedup vs baseline: 1.1334x; 1.1334x over previous
import functools

import jax
import jax.numpy as jnp
from jax import lax
from jax.experimental import pallas as pl
from jax.experimental.pallas import tpu as pltpu
from jax.experimental.pallas import tpu_sc as plsc

H, B, N, D = 3, 4, 2048, 2048
K = N // 2
HB = H * B
BN = 512
NBLK = N // BN
SIGN = -2 ** 31
L = 16
CHUNKS = N // L
UNROLL = 8


def _tc_body(x_ref, w_ref, xout_ref, score_out_ref, score_ref):
    hb = pl.program_id(0)
    jn = pl.program_id(1)

    xraw = x_ref[...]
    xout_ref[...] = xraw

    x = xraw[0].astype(jnp.bfloat16).astype(jnp.float32)
    w = w_ref[...].astype(jnp.bfloat16).astype(jnp.float32)
    s = jax.lax.dot_general(x, w, (((1,), (1,)), ((), ())),
                            preferred_element_type=jnp.float32)
    score_ref[pl.ds(hb, 1), pl.ds(jn * BN, BN)] = s.reshape(1, BN)

    @pl.when(jnp.logical_and(hb == HB - 1, jn == NBLK - 1))
    def _():
        score_out_ref[...] = score_ref[...]


def _tc_scores(x, W):
    return pl.pallas_call(
        _tc_body,
        grid=(HB, NBLK),
        in_specs=[
            pl.BlockSpec((1, BN, D), lambda i, j: (i, j, 0)),
            pl.BlockSpec((1, D), lambda i, j: (0, 0)),
        ],
        out_specs=[
            pl.BlockSpec((1, BN, D), lambda i, j: (i, j, 0)),
            pl.BlockSpec((HB, N), lambda i, j: (0, 0)),
        ],
        out_shape=[
            jax.ShapeDtypeStruct((HB, N, D), jnp.float32),
            jax.ShapeDtypeStruct((HB, N), jnp.float32),
        ],
        scratch_shapes=[pltpu.VMEM((HB, N), jnp.float32)],
        compiler_params=pltpu.CompilerParams(
            dimension_semantics=("arbitrary", "arbitrary")),
    )(x, W)


def _sc_select_body(scores_hbm, sel_hbm, row_v, key_v, sel_v):
    c = lax.axis_index("c")
    s = lax.axis_index("s")

    @pl.when(jnp.logical_and(c == 0, s < HB))
    def _select_row():
        pltpu.sync_copy(scores_hbm.at[s], row_v)

        def mk(i, _):
            for u in range(UNROLL):
                off = (i * UNROLL + u) * L
                sv = row_v[pl.ds(off, L)]
                sv = jnp.where(sv == 0.0, 0.0, sv)
                bv = lax.bitcast_convert_type(sv, jnp.int32)
                key_v[pl.ds(off, L)] = jnp.where(
                    bv >= 0, bv, bv ^ jnp.int32(0x7FFFFFFF))
            return 0
        lax.fori_loop(0, CHUNKS // UNROLL, mk, 0, unroll=False)

        sign = jnp.int32(SIGN)
        zeros = jnp.zeros((L,), jnp.int32)

        def count_pred(pred_of_chunk):
            def inner(j, av):
                for u in range(UNROLL):
                    off = (j * UNROLL + u) * L
                    av = av + plsc.all_reduce_population_count(
                        pred_of_chunk(off))
                return av
            return lax.fori_loop(0, CHUNKS // UNROLL, inner, zeros,
                                 unroll=False)

        def count_lt(t_s):
            return count_pred(lambda off: key_v[pl.ds(off, L)] < t_s)

        def bit_pass(i, acc):
            t = acc | lax.shift_left(jnp.int32(1), 31 - i)
            cnt = count_lt(t ^ sign)
            return jnp.where(cnt < K, t, acc)
        acc = lax.fori_loop(0, 32, bit_pass, zeros, unroll=False)
        vs = acc ^ sign
        r = K - count_lt(vs)

        lane = lax.iota(jnp.int32, L)

        def idx_pass(i, acc2):
            t = acc2 | lax.shift_left(jnp.int32(1), 10 - i)
            cnt = count_pred(
                lambda off: (key_v[pl.ds(off, L)] == vs) & ((lane + off) < t))
            return jnp.where(cnt < r, t, acc2)
        istar = lax.fori_loop(0, 11, idx_pass, zeros, unroll=False)

        def sel_pass(j, _):
            for u in range(UNROLL):
                off = (j * UNROLL + u) * L
                kv = key_v[pl.ds(off, L)]
                keep = (kv < vs) | ((kv == vs) & ((lane + off) <= istar))
                sel_v[pl.ds(off, L)] = jnp.where(keep, 1.0, 0.0)
            return 0
        lax.fori_loop(0, CHUNKS // UNROLL, sel_pass, 0, unroll=False)
        pltpu.sync_copy(sel_v, sel_hbm.at[s])


def _sc_combine_body(sel_hbm, src_hbm, out_hbm, h0_v, h1_v, h2_v, src_v, out_v):
    c = lax.axis_index("c")
    s = lax.axis_index("s")

    @pl.when(jnp.logical_and(c == 0, s < B))
    def _combine():
        pltpu.sync_copy(sel_hbm.at[s], h0_v)
        pltpu.sync_copy(sel_hbm.at[B + s], h1_v)
        pltpu.sync_copy(sel_hbm.at[2 * B + s], h2_v)
        pltpu.sync_copy(src_hbm.at[s], src_v)

        def comb(i, _):
            for u in range(UNROLL):
                off = (i * UNROLL + u) * L
                cnt = (h0_v[pl.ds(off, L)] + h1_v[pl.ds(off, L)]
                       + h2_v[pl.ds(off, L)])
                out_v[pl.ds(off, L)] = cnt + (float(H) - cnt) * src_v[pl.ds(off, L)]
            return 0
        lax.fori_loop(0, CHUNKS // UNROLL, comb, 0, unroll=False)
        pltpu.sync_copy(out_v, out_hbm.at[s])


_SC_MESH = plsc.VectorSubcoreMesh(core_axis_name="c", subcore_axis_name="s")

_sc_select = functools.partial(
    pl.kernel,
    out_type=jax.ShapeDtypeStruct((HB, N), jnp.float32),
    mesh=_SC_MESH,
    scratch_types=[
        pltpu.VMEM((N,), jnp.float32),
        pltpu.VMEM((N,), jnp.int32),
        pltpu.VMEM((N,), jnp.float32),
    ],
    compiler_params=pltpu.CompilerParams(needs_layout_passes=False),
)(_sc_select_body)

_sc_combine = functools.partial(
    pl.kernel,
    out_type=jax.ShapeDtypeStruct((B, N), jnp.float32),
    mesh=_SC_MESH,
    scratch_types=[
        pltpu.VMEM((N,), jnp.float32),
        pltpu.VMEM((N,), jnp.float32),
        pltpu.VMEM((N,), jnp.float32),
        pltpu.VMEM((N,), jnp.float32),
        pltpu.VMEM((N,), jnp.float32),
    ],
    compiler_params=pltpu.CompilerParams(needs_layout_passes=False),
)(_sc_combine_body)


def kernel(x_list, src_mask, W, b):
    x = x_list.reshape(HB, N, D)
    src = src_mask.reshape(B, N)
    x_out, scores = _tc_scores(x, W)
    sel = _sc_select(scores)
    mask = _sc_combine(sel, src)
    return x_out.reshape(H, B, N, D), mask.reshape(B, 1, N)


# speedup vs baseline: 1.1530x; 1.0173x over previous
import functools

import jax
import jax.numpy as jnp
from jax import lax
from jax.experimental import pallas as pl
from jax.experimental.pallas import tpu as pltpu
from jax.experimental.pallas import tpu_sc as plsc

H, B, N, D = 3, 4, 2048, 2048
K = N // 2
HB = H * B
BN = 1024
NBLK = N // BN
SIGN = -2 ** 31
L = 16
CHUNKS = N // L
UNROLL = 8


def _tc_body(x_ref, w_ref, xout_ref, score_out_ref, score_ref):
    hb = pl.program_id(0)
    jn = pl.program_id(1)

    xraw = x_ref[...]
    xout_ref[...] = xraw

    x = xraw[0].astype(jnp.bfloat16).astype(jnp.float32)
    w = w_ref[...].astype(jnp.bfloat16).astype(jnp.float32)
    s = jax.lax.dot_general(x, w, (((1,), (1,)), ((), ())),
                            preferred_element_type=jnp.float32)
    score_ref[pl.ds(hb, 1), pl.ds(jn * BN, BN)] = s.reshape(1, BN)

    @pl.when(jnp.logical_and(hb == HB - 1, jn == NBLK - 1))
    def _():
        score_out_ref[...] = score_ref[...]


def _tc_scores(x, W):
    return pl.pallas_call(
        _tc_body,
        grid=(HB, NBLK),
        in_specs=[
            pl.BlockSpec((1, BN, D), lambda i, j: (i, j, 0)),
            pl.BlockSpec((1, D), lambda i, j: (0, 0)),
        ],
        out_specs=[
            pl.BlockSpec((1, BN, D), lambda i, j: (i, j, 0)),
            pl.BlockSpec((HB, N), lambda i, j: (0, 0)),
        ],
        out_shape=[
            jax.ShapeDtypeStruct((HB, N, D), jnp.float32),
            jax.ShapeDtypeStruct((HB, N), jnp.float32),
        ],
        scratch_shapes=[pltpu.VMEM((HB, N), jnp.float32)],
        compiler_params=pltpu.CompilerParams(
            dimension_semantics=("arbitrary", "arbitrary")),
    )(x, W)


def _sc_select_body(scores_hbm, sel_hbm, row_v, key_v, sel_v):
    c = lax.axis_index("c")
    s = lax.axis_index("s")

    @pl.when(jnp.logical_and(c == 0, s < HB))
    def _select_row():
        pltpu.sync_copy(scores_hbm.at[s], row_v)

        def mk(i, _):
            for u in range(UNROLL):
                off = (i * UNROLL + u) * L
                sv = row_v[pl.ds(off, L)]
                sv = jnp.where(sv == 0.0, 0.0, sv)
                bv = lax.bitcast_convert_type(sv, jnp.int32)
                key_v[pl.ds(off, L)] = jnp.where(
                    bv >= 0, bv, bv ^ jnp.int32(0x7FFFFFFF))
            return 0
        lax.fori_loop(0, CHUNKS // UNROLL, mk, 0, unroll=False)

        sign = jnp.int32(SIGN)
        zeros = jnp.zeros((L,), jnp.int32)

        def count_pred(pred_of_chunk):
            def inner(j, av):
                for u in range(UNROLL):
                    off = (j * UNROLL + u) * L
                    av = av + plsc.all_reduce_population_count(
                        pred_of_chunk(off))
                return av
            return lax.fori_loop(0, CHUNKS // UNROLL, inner, zeros,
                                 unroll=False)

        def count_lt(t_s):
            return count_pred(lambda off: key_v[pl.ds(off, L)] < t_s)

        def bit_pass(i, acc):
            t = acc | lax.shift_left(jnp.int32(1), 31 - i)
            cnt = count_lt(t ^ sign)
            return jnp.where(cnt < K, t, acc)
        acc = lax.fori_loop(0, 32, bit_pass, zeros, unroll=False)
        vs = acc ^ sign
        r = K - count_lt(vs)

        lane = lax.iota(jnp.int32, L)

        def idx_pass(i, acc2):
            t = acc2 | lax.shift_left(jnp.int32(1), 10 - i)
            cnt = count_pred(
                lambda off: (key_v[pl.ds(off, L)] == vs) & ((lane + off) < t))
            return jnp.where(cnt < r, t, acc2)
        istar = lax.fori_loop(0, 11, idx_pass, zeros, unroll=False)

        def sel_pass(j, _):
            for u in range(UNROLL):
                off = (j * UNROLL + u) * L
                kv = key_v[pl.ds(off, L)]
                keep = (kv < vs) | ((kv == vs) & ((lane + off) <= istar))
                sel_v[pl.ds(off, L)] = jnp.where(keep, 1.0, 0.0)
            return 0
        lax.fori_loop(0, CHUNKS // UNROLL, sel_pass, 0, unroll=False)
        pltpu.sync_copy(sel_v, sel_hbm.at[s])


def _sc_combine_body(sel_hbm, src_hbm, out_hbm, h0_v, h1_v, h2_v, src_v, out_v):
    c = lax.axis_index("c")
    s = lax.axis_index("s")

    @pl.when(jnp.logical_and(c == 0, s < B))
    def _combine():
        pltpu.sync_copy(sel_hbm.at[s], h0_v)
        pltpu.sync_copy(sel_hbm.at[B + s], h1_v)
        pltpu.sync_copy(sel_hbm.at[2 * B + s], h2_v)
        pltpu.sync_copy(src_hbm.at[s], src_v)

        def comb(i, _):
            for u in range(UNROLL):
                off = (i * UNROLL + u) * L
                cnt = (h0_v[pl.ds(off, L)] + h1_v[pl.ds(off, L)]
                       + h2_v[pl.ds(off, L)])
                out_v[pl.ds(off, L)] = cnt + (float(H) - cnt) * src_v[pl.ds(off, L)]
            return 0
        lax.fori_loop(0, CHUNKS // UNROLL, comb, 0, unroll=False)
        pltpu.sync_copy(out_v, out_hbm.at[s])


_SC_MESH = plsc.VectorSubcoreMesh(core_axis_name="c", subcore_axis_name="s")

_sc_select = functools.partial(
    pl.kernel,
    out_type=jax.ShapeDtypeStruct((HB, N), jnp.float32),
    mesh=_SC_MESH,
    scratch_types=[
        pltpu.VMEM((N,), jnp.float32),
        pltpu.VMEM((N,), jnp.int32),
        pltpu.VMEM((N,), jnp.float32),
    ],
    compiler_params=pltpu.CompilerParams(needs_layout_passes=False),
)(_sc_select_body)

_sc_combine = functools.partial(
    pl.kernel,
    out_type=jax.ShapeDtypeStruct((B, N), jnp.float32),
    mesh=_SC_MESH,
    scratch_types=[
        pltpu.VMEM((N,), jnp.float32),
        pltpu.VMEM((N,), jnp.float32),
        pltpu.VMEM((N,), jnp.float32),
        pltpu.VMEM((N,), jnp.float32),
        pltpu.VMEM((N,), jnp.float32),
    ],
    compiler_params=pltpu.CompilerParams(needs_layout_passes=False),
)(_sc_combine_body)


def kernel(x_list, src_mask, W, b):
    x = x_list.reshape(HB, N, D)
    src = src_mask.reshape(B, N)
    x_out, scores = _tc_scores(x, W)
    sel = _sc_select(scores)
    mask = _sc_combine(sel, src)
    return x_out.reshape(H, B, N, D), mask.reshape(B, 1, N)
